# baseline (device time: 167059 ns/iter reference)
import jax
import jax.numpy as jnp
from jax import lax
from jax.experimental import pallas as pl
from jax.experimental.pallas import tpu as pltpu

N_DEV = 4


def _allreduce_body(part_ref, out_ref, acc_ref, rbuf_ref, send_sems, recv_sems):
    my = lax.axis_index("i")
    p1 = my ^ 1
    p2 = 3 - my

    barrier = pltpu.get_barrier_semaphore()
    for nbr in (p1, p2):
        pl.semaphore_signal(
            barrier, inc=1, device_id=(nbr,),
            device_id_type=pl.DeviceIdType.MESH,
        )
    pl.semaphore_wait(barrier, 2)

    h = (my ^ (my >> 1)) & 1
    kq_loc = my >> 1
    kh = h * 2
    sh = (1 - h) * 2
    kq = kh + kq_loc
    sq = kh + (1 - kq_loc)

    acc_ref[...] = part_ref[...]

    r1 = pltpu.make_async_remote_copy(
        src_ref=acc_ref.at[pl.ds(sh, 2)],
        dst_ref=rbuf_ref.at[pl.ds(0, 2)],
        send_sem=send_sems.at[0], recv_sem=recv_sems.at[0],
        device_id=(p1,), device_id_type=pl.DeviceIdType.MESH,
    )
    r1.start()
    r1.wait()
    acc_ref[pl.ds(kh, 2)] = acc_ref[pl.ds(kh, 2)] + rbuf_ref[pl.ds(0, 2)]

    r2 = pltpu.make_async_remote_copy(
        src_ref=acc_ref.at[pl.ds(sq, 1)],
        dst_ref=rbuf_ref.at[pl.ds(2, 1)],
        send_sem=send_sems.at[1], recv_sem=recv_sems.at[1],
        device_id=(p2,), device_id_type=pl.DeviceIdType.MESH,
    )
    r2.start()
    r2.wait()
    acc_ref[pl.ds(kq, 1)] = acc_ref[pl.ds(kq, 1)] + rbuf_ref[pl.ds(2, 1)]

    r3 = pltpu.make_async_remote_copy(
        src_ref=acc_ref.at[pl.ds(kq, 1)],
        dst_ref=rbuf_ref.at[pl.ds(3, 1)],
        send_sem=send_sems.at[2], recv_sem=recv_sems.at[2],
        device_id=(p2,), device_id_type=pl.DeviceIdType.MESH,
    )
    r3.start()
    r3.wait()
    acc_ref[pl.ds(sq, 1)] = rbuf_ref[pl.ds(3, 1)]

    r4 = pltpu.make_async_remote_copy(
        src_ref=acc_ref.at[pl.ds(kh, 2)],
        dst_ref=rbuf_ref.at[pl.ds(4, 2)],
        send_sem=send_sems.at[3], recv_sem=recv_sems.at[3],
        device_id=(p1,), device_id_type=pl.DeviceIdType.MESH,
    )
    r4.start()
    r4.wait()
    acc_ref[pl.ds(sh, 2)] = rbuf_ref[pl.ds(4, 2)]

    out_ref[...] = acc_ref[...].astype(jnp.float32)


def kernel(ids, E):
    v_shard, d = E.shape
    t = ids.shape[0]
    q = t // N_DEV

    my = lax.axis_index("i")
    loc = ids - my * v_shard
    mask = (loc >= 0) & (loc < v_shard)
    safe = jnp.where(mask, loc, 0)
    part = jnp.where(mask[:, None], E[safe, :], 0.0).astype(jnp.bfloat16)
    part = part.reshape(N_DEV, q, d)

    out = pl.pallas_call(
        _allreduce_body,
        out_shape=jax.ShapeDtypeStruct((N_DEV, q, d), jnp.float32),
        in_specs=[pl.BlockSpec(memory_space=pltpu.VMEM)],
        out_specs=pl.BlockSpec(memory_space=pltpu.VMEM),
        scratch_shapes=[
            pltpu.VMEM((N_DEV, q, d), jnp.bfloat16),
            pltpu.VMEM((6, q, d), jnp.bfloat16),
            pltpu.SemaphoreType.DMA((4,)),
            pltpu.SemaphoreType.DMA((4,)),
        ],
        compiler_params=pltpu.CompilerParams(collective_id=0),
    )(part)
    return out.reshape(t, d)


# device time: 143574 ns/iter; 1.1636x vs baseline; 1.1636x over previous
import jax
import jax.numpy as jnp
from jax import lax
from jax.experimental import pallas as pl
from jax.experimental.pallas import tpu as pltpu

N_DEV = 4
T = 2048
D = 1024
V_SHARD = 16384
Q = T // 4
H = T // 2
UNROLL = 8


def _body(ids_sref, E_ref, mask_ref, out_ref,
          stage_ref, acc_ref, rbuf_ref, gsem, send_sems, recv_sems):
    my = lax.axis_index("i")
    p1 = my ^ 1
    p2 = 3 - my
    base = my * V_SHARD

    def issue(i):
        idx = jnp.clip(ids_sref[i] - base, 0, V_SHARD - 1)
        pltpu.make_async_copy(
            E_ref.at[pl.ds(idx, 1)], stage_ref.at[pl.ds(i, 1)], gsem
        ).start()

    def step(s, carry):
        for u in range(UNROLL):
            issue(s * UNROLL + u)
        return carry

    lax.fori_loop(0, T // UNROLL, step, 0)

    barrier = pltpu.get_barrier_semaphore()
    for nbr in (p1, p2):
        pl.semaphore_signal(
            barrier, inc=1, device_id=(nbr,),
            device_id_type=pl.DeviceIdType.MESH,
        )
    pl.semaphore_wait(barrier, 2)

    pltpu.make_async_copy(E_ref.at[pl.ds(0, T)], stage_ref, gsem).wait()

    acc_ref[...] = (stage_ref[...] * mask_ref[...]).astype(jnp.bfloat16)

    h = (my ^ (my >> 1)) & 1
    kq_loc = my >> 1
    kh = h * H
    sh = (1 - h) * H
    kq = kh + kq_loc * Q
    sq = kh + (1 - kq_loc) * Q

    r1 = pltpu.make_async_remote_copy(
        src_ref=acc_ref.at[pl.ds(sh, H)],
        dst_ref=rbuf_ref.at[pl.ds(0, H)],
        send_sem=send_sems.at[0], recv_sem=recv_sems.at[0],
        device_id=(p1,), device_id_type=pl.DeviceIdType.MESH,
    )
    r1.start()
    r1.wait()
    acc_ref[pl.ds(kh, H)] = acc_ref[pl.ds(kh, H)] + rbuf_ref[pl.ds(0, H)]

    r2 = pltpu.make_async_remote_copy(
        src_ref=acc_ref.at[pl.ds(sq, Q)],
        dst_ref=rbuf_ref.at[pl.ds(H, Q)],
        send_sem=send_sems.at[1], recv_sem=recv_sems.at[1],
        device_id=(p2,), device_id_type=pl.DeviceIdType.MESH,
    )
    r2.start()
    r2.wait()
    acc_ref[pl.ds(kq, Q)] = acc_ref[pl.ds(kq, Q)] + rbuf_ref[pl.ds(H, Q)]

    r3 = pltpu.make_async_remote_copy(
        src_ref=acc_ref.at[pl.ds(kq, Q)],
        dst_ref=rbuf_ref.at[pl.ds(H + Q, Q)],
        send_sem=send_sems.at[2], recv_sem=recv_sems.at[2],
        device_id=(p2,), device_id_type=pl.DeviceIdType.MESH,
    )
    r3.start()
    r3.wait()
    acc_ref[pl.ds(sq, Q)] = rbuf_ref[pl.ds(H + Q, Q)]

    r4 = pltpu.make_async_remote_copy(
        src_ref=acc_ref.at[pl.ds(kh, H)],
        dst_ref=rbuf_ref.at[pl.ds(T, H)],
        send_sem=send_sems.at[3], recv_sem=recv_sems.at[3],
        device_id=(p1,), device_id_type=pl.DeviceIdType.MESH,
    )
    r4.start()
    r4.wait()
    acc_ref[pl.ds(sh, H)] = rbuf_ref[pl.ds(T, H)]

    out_ref[...] = acc_ref[...].astype(jnp.float32)


def kernel(ids, E):
    my = lax.axis_index("i")
    base = my * V_SHARD
    mask = ((ids >= base) & (ids < base + V_SHARD)).astype(jnp.float32)[:, None]

    grid_spec = pltpu.PrefetchScalarGridSpec(
        num_scalar_prefetch=1,
        grid=(1,),
        in_specs=[
            pl.BlockSpec(memory_space=pl.ANY),
            pl.BlockSpec(memory_space=pltpu.VMEM),
        ],
        out_specs=pl.BlockSpec(memory_space=pltpu.VMEM),
        scratch_shapes=[
            pltpu.VMEM((T, D), jnp.float32),
            pltpu.VMEM((T, D), jnp.bfloat16),
            pltpu.VMEM((T + H, D), jnp.bfloat16),
            pltpu.SemaphoreType.DMA,
            pltpu.SemaphoreType.DMA((4,)),
            pltpu.SemaphoreType.DMA((4,)),
        ],
    )
    return pl.pallas_call(
        _body,
        grid_spec=grid_spec,
        out_shape=jax.ShapeDtypeStruct((T, D), jnp.float32),
        compiler_params=pltpu.CompilerParams(collective_id=0),
    )(ids, E, mask)


# device time: 86765 ns/iter; 1.9254x vs baseline; 1.6547x over previous
import jax
import jax.numpy as jnp
from jax import lax
from jax.experimental import pallas as pl
from jax.experimental.pallas import tpu as pltpu

N_DEV = 4
T = 2048
D = 1024
V_SHARD = 16384
HS = T // 4
QS = T // 8
UNROLL = 8


def _mesh_id(dev):
    return (dev,)


def _body(loc_sref, E_ref, out_ref, stage_ref, acc_ref, rbuf_ref,
          gsem, send_sems, recv_sems):
    my = lax.axis_index("i")
    p1 = my ^ 1
    p2 = 3 - my

    stage_ref[...] = jnp.zeros_like(stage_ref)

    barrier = pltpu.get_barrier_semaphore()
    for nbr in (p1, p2):
        pl.semaphore_signal(
            barrier, inc=1, device_id=(nbr,),
            device_id_type=pl.DeviceIdType.MESH,
        )
    pl.semaphore_wait(barrier, 2)

    def step(s, n):
        for u in range(UNROLL):
            i = s * UNROLL + u
            idx = loc_sref[i]
            owned = (idx >= 0) & (idx < V_SHARD)

            @pl.when(owned)
            def _():
                pltpu.make_async_copy(
                    E_ref.at[pl.ds(idx, 1)], stage_ref.at[pl.ds(i, 1)], gsem
                ).start()

            n = n + owned.astype(jnp.int32)
        return n

    n_owned = lax.fori_loop(0, T // UNROLL, step, jnp.int32(0))

    def drain(_, carry):
        pltpu.make_async_copy(
            E_ref.at[pl.ds(0, 1)], stage_ref.at[pl.ds(0, 1)], gsem
        ).wait()
        return carry

    lax.fori_loop(0, n_owned, drain, jnp.int32(0))

    acc_ref[...] = stage_ref[...].astype(jnp.bfloat16)

    hA = (my ^ (my >> 1)) & 1
    qA = my >> 1
    hB = my >> 1
    qB = my & 1

    khA = hA * HS
    shA = (1 - hA) * HS
    kqA = khA + qA * QS
    sqA = khA + (1 - qA) * QS
    khB = 2 * HS + hB * HS
    shB = 2 * HS + (1 - hB) * HS
    kqB = khB + qB * QS
    sqB = khB + (1 - qB) * QS

    def exch(sem_idx, src_off, n_rows, rbuf_off, peer):
        return pltpu.make_async_remote_copy(
            src_ref=acc_ref.at[pl.ds(src_off, n_rows)],
            dst_ref=rbuf_ref.at[pl.ds(rbuf_off, n_rows)],
            send_sem=send_sems.at[sem_idx], recv_sem=recv_sems.at[sem_idx],
            device_id=(peer,), device_id_type=pl.DeviceIdType.MESH,
        )

    a1 = exch(0, shA, HS, 0, p1)
    b1 = exch(1, shB, HS, HS, p2)
    a1.start()
    b1.start()
    a1.wait()
    acc_ref[pl.ds(khA, HS)] = acc_ref[pl.ds(khA, HS)] + rbuf_ref[pl.ds(0, HS)]
    b1.wait()
    acc_ref[pl.ds(khB, HS)] = acc_ref[pl.ds(khB, HS)] + rbuf_ref[pl.ds(HS, HS)]

    a2 = exch(2, sqA, QS, 2 * HS, p2)
    b2 = exch(3, sqB, QS, 2 * HS + QS, p1)
    a2.start()
    b2.start()
    a2.wait()
    acc_ref[pl.ds(kqA, QS)] = (
        acc_ref[pl.ds(kqA, QS)] + rbuf_ref[pl.ds(2 * HS, QS)]
    )
    b2.wait()
    acc_ref[pl.ds(kqB, QS)] = (
        acc_ref[pl.ds(kqB, QS)] + rbuf_ref[pl.ds(2 * HS + QS, QS)]
    )

    a3 = exch(4, kqA, QS, 3 * HS, p2)
    b3 = exch(5, kqB, QS, 3 * HS + QS, p1)
    a3.start()
    b3.start()
    a3.wait()
    acc_ref[pl.ds(sqA, QS)] = rbuf_ref[pl.ds(3 * HS, QS)]
    b3.wait()
    acc_ref[pl.ds(sqB, QS)] = rbuf_ref[pl.ds(3 * HS + QS, QS)]

    a4 = exch(6, khA, HS, 4 * HS, p1)
    b4 = exch(7, khB, HS, 5 * HS, p2)
    a4.start()
    b4.start()
    a4.wait()
    acc_ref[pl.ds(shA, HS)] = rbuf_ref[pl.ds(4 * HS, HS)]
    b4.wait()
    acc_ref[pl.ds(shB, HS)] = rbuf_ref[pl.ds(5 * HS, HS)]

    out_ref[...] = acc_ref[...].astype(jnp.float32)


def kernel(ids, E):
    my = lax.axis_index("i")
    loc = (ids - my * V_SHARD).astype(jnp.int32)

    grid_spec = pltpu.PrefetchScalarGridSpec(
        num_scalar_prefetch=1,
        grid=(1,),
        in_specs=[
            pl.BlockSpec(memory_space=pl.ANY),
        ],
        out_specs=pl.BlockSpec(memory_space=pltpu.VMEM),
        scratch_shapes=[
            pltpu.VMEM((T, D), jnp.float32),
            pltpu.VMEM((T, D), jnp.bfloat16),
            pltpu.VMEM((6 * HS, D), jnp.bfloat16),
            pltpu.SemaphoreType.DMA,
            pltpu.SemaphoreType.DMA((8,)),
            pltpu.SemaphoreType.DMA((8,)),
        ],
    )
    return pl.pallas_call(
        _body,
        grid_spec=grid_spec,
        out_shape=jax.ShapeDtypeStruct((T, D), jnp.float32),
        compiler_params=pltpu.CompilerParams(collective_id=0),
    )(loc, E)


# device time: 74306 ns/iter; 2.2483x vs baseline; 1.1677x over previous
import jax
import jax.numpy as jnp
from jax import lax
from jax.experimental import pallas as pl
from jax.experimental.pallas import tpu as pltpu

N_DEV = 4
T = 2048
D = 1024
V_SHARD = 16384
HS = T // 4
QS = T // 8
UNROLL = 8


def _body(loc_sref, E_ref, out_ref, stage_ref, acc_ref, rbuf_ref,
          gsems, send_sems, recv_sems):
    my = lax.axis_index("i")
    p1 = my ^ 1
    p2 = 3 - my

    stage_ref[...] = jnp.zeros_like(stage_ref)

    barrier = pltpu.get_barrier_semaphore()
    for nbr in (p1, p2):
        pl.semaphore_signal(
            barrier, inc=1, device_id=(nbr,),
            device_id_type=pl.DeviceIdType.MESH,
        )
    pl.semaphore_wait(barrier, 2)

    def gather_chunk(start_row, c):
        def step(s, n):
            for u in range(UNROLL):
                i = start_row + s * UNROLL + u
                idx = loc_sref[i]
                owned = (idx >= 0) & (idx < V_SHARD)

                @pl.when(owned)
                def _():
                    pltpu.make_async_copy(
                        E_ref.at[pl.ds(idx, 1)],
                        stage_ref.at[pl.ds(i, 1)],
                        gsems.at[c],
                    ).start()

                n = n + owned.astype(jnp.int32)
            return n

        return lax.fori_loop(0, HS // UNROLL, step, jnp.int32(0))

    def drain_and_convert(n, c, start_row):
        def wait_one(_, carry):
            pltpu.make_async_copy(
                E_ref.at[pl.ds(0, 1)], stage_ref.at[pl.ds(0, 1)], gsems.at[c]
            ).wait()
            return carry

        lax.fori_loop(0, n, wait_one, jnp.int32(0))
        acc_ref[pl.ds(start_row, HS)] = (
            stage_ref[pl.ds(start_row, HS)].astype(jnp.bfloat16)
        )

    hA = (my ^ (my >> 1)) & 1
    qA = my >> 1
    hB = my >> 1
    qB = my & 1

    khA = hA * HS
    shA = (1 - hA) * HS
    kqA = khA + qA * QS
    sqA = khA + (1 - qA) * QS
    khB = 2 * HS + hB * HS
    shB = 2 * HS + (1 - hB) * HS
    kqB = khB + qB * QS
    sqB = khB + (1 - qB) * QS

    def exch(sem_idx, src_off, n_rows, rbuf_off, peer):
        return pltpu.make_async_remote_copy(
            src_ref=acc_ref.at[pl.ds(src_off, n_rows)],
            dst_ref=rbuf_ref.at[pl.ds(rbuf_off, n_rows)],
            send_sem=send_sems.at[sem_idx], recv_sem=recv_sems.at[sem_idx],
            device_id=(peer,), device_id_type=pl.DeviceIdType.MESH,
        )

    nA = gather_chunk(shA, 0)
    drain_and_convert(nA, 0, shA)
    a1 = exch(0, shA, HS, 0, p1)
    a1.start()

    nB = gather_chunk(shB, 1)
    drain_and_convert(nB, 1, shB)
    b1 = exch(1, shB, HS, HS, p2)
    b1.start()

    nKA = gather_chunk(khA, 2)
    drain_and_convert(nKA, 2, khA)

    a1.wait()
    acc_ref[pl.ds(khA, HS)] = acc_ref[pl.ds(khA, HS)] + rbuf_ref[pl.ds(0, HS)]
    a2 = exch(2, sqA, QS, 2 * HS, p2)
    a2.start()

    nKB = gather_chunk(khB, 3)
    drain_and_convert(nKB, 3, khB)

    b1.wait()
    acc_ref[pl.ds(khB, HS)] = acc_ref[pl.ds(khB, HS)] + rbuf_ref[pl.ds(HS, HS)]
    b2 = exch(3, sqB, QS, 2 * HS + QS, p1)
    b2.start()

    a2.wait()
    acc_ref[pl.ds(kqA, QS)] = (
        acc_ref[pl.ds(kqA, QS)] + rbuf_ref[pl.ds(2 * HS, QS)]
    )
    a3 = exch(4, kqA, QS, 3 * HS, p2)
    a3.start()

    b2.wait()
    acc_ref[pl.ds(kqB, QS)] = (
        acc_ref[pl.ds(kqB, QS)] + rbuf_ref[pl.ds(2 * HS + QS, QS)]
    )
    b3 = exch(5, kqB, QS, 3 * HS + QS, p1)
    b3.start()

    a3.wait()
    acc_ref[pl.ds(sqA, QS)] = rbuf_ref[pl.ds(3 * HS, QS)]
    a4 = exch(6, khA, HS, 4 * HS, p1)
    a4.start()

    b3.wait()
    acc_ref[pl.ds(sqB, QS)] = rbuf_ref[pl.ds(3 * HS + QS, QS)]
    b4 = exch(7, khB, HS, 5 * HS, p2)
    b4.start()

    a4.wait()
    acc_ref[pl.ds(shA, HS)] = rbuf_ref[pl.ds(4 * HS, HS)]
    b4.wait()
    acc_ref[pl.ds(shB, HS)] = rbuf_ref[pl.ds(5 * HS, HS)]

    out_ref[...] = acc_ref[...].astype(jnp.float32)


def kernel(ids, E):
    my = lax.axis_index("i")
    loc = (ids - my * V_SHARD).astype(jnp.int32)

    grid_spec = pltpu.PrefetchScalarGridSpec(
        num_scalar_prefetch=1,
        grid=(1,),
        in_specs=[
            pl.BlockSpec(memory_space=pl.ANY),
        ],
        out_specs=pl.BlockSpec(memory_space=pltpu.VMEM),
        scratch_shapes=[
            pltpu.VMEM((T, D), jnp.float32),
            pltpu.VMEM((T, D), jnp.bfloat16),
            pltpu.VMEM((6 * HS, D), jnp.bfloat16),
            pltpu.SemaphoreType.DMA((4,)),
            pltpu.SemaphoreType.DMA((8,)),
            pltpu.SemaphoreType.DMA((8,)),
        ],
    )
    return pl.pallas_call(
        _body,
        grid_spec=grid_spec,
        out_shape=jax.ShapeDtypeStruct((T, D), jnp.float32),
        compiler_params=pltpu.CompilerParams(collective_id=0),
    )(loc, E)
